# Initial kernel scaffold; baseline (speedup 1.0000x reference)
import jax
import jax.numpy as jnp
from jax.experimental import pallas as pl


def kernel(x, edge_index, batch, layers, W1, b1, W2, b2, W3, b3, Wl, bl):
    raise NotImplementedError("write your pallas kernel here")


# speedup vs baseline: 19.4167x; 19.4167x over previous
import functools

import jax
import jax.numpy as jnp
from jax import lax
from jax.experimental import pallas as pl
from jax.experimental.pallas import tpu as pltpu
from jax.experimental.pallas import tpu_sc as plsc

N = 10000
D = 128
E = 320000
G_OUT = 64
CHUNK = 128
N_CHUNKS = E // CHUNK
NC, NS = 2, 16
NW = NC * NS
ROWS_PER_TILE = N // NS
R = 2000
N_BLOCKS = N // R

_mesh = plsc.VectorSubcoreMesh(core_axis_name="c", subcore_axis_name="s")


N_PAD = 10240
SLAB = N_PAD // NS


@functools.partial(
    pl.kernel,
    out_type=jax.ShapeDtypeStruct((NC, N_PAD), jnp.float32),
    mesh=_mesh,
    scratch_types=[
        pltpu.VMEM((CHUNK,), jnp.int32),
        pltpu.VMEM((CHUNK,), jnp.float32),
        pltpu.VMEM_SHARED((N_PAD,), jnp.float32),
    ],
)
def _deg(edges_hbm, out_hbm, idx_v, val_v, hacc):
    cid = lax.axis_index("c")
    sid = lax.axis_index("s")
    wid = sid * NC + cid

    zeros = jnp.zeros((16,), jnp.float32)

    def zbody(i, carry):
        val_v[pl.ds(i * 16, 16)] = zeros
        return carry

    lax.fori_loop(0, CHUNK // 16, zbody, 0)
    base = sid * SLAB
    for k in range(SLAB // CHUNK):
        pltpu.sync_copy(val_v, hacc.at[pl.ds(base + k * CHUNK, CHUNK)])
    plsc.subcore_barrier()

    ones = jnp.ones((16,), jnp.float32)

    def obody(i, carry):
        val_v[pl.ds(i * 16, 16)] = ones
        return carry

    lax.fori_loop(0, CHUNK // 16, obody, 0)

    n_my = jnp.where(wid < N_CHUNKS % NW, N_CHUNKS // NW + 1, N_CHUNKS // NW)

    def body(t, carry):
        c = wid + t * NW
        pltpu.sync_copy(edges_hbm.at[1, pl.ds(c * CHUNK, CHUNK)], idx_v)
        pltpu.sync_copy(val_v, hacc.at[idx_v], add=True)
        return carry

    lax.fori_loop(0, n_my, body, 0)
    plsc.subcore_barrier()
    pltpu.sync_copy(hacc.at[pl.ds(base, SLAB)],
                    out_hbm.at[cid, pl.ds(base, SLAB)])


@functools.partial(
    pl.kernel,
    out_type=jax.ShapeDtypeStruct((NC, N, D), jnp.float32),
    mesh=_mesh,
    scratch_types=[
        pltpu.VMEM((CHUNK, D), jnp.float32),
        pltpu.VMEM((CHUNK,), jnp.int32),
        pltpu.VMEM((CHUNK,), jnp.int32),
        pltpu.VMEM_SHARED((N_PAD, D), jnp.float32),
        pltpu.SemaphoreType.DMA,
    ],
)
def _prop(g_hbm, edges_hbm, out_hbm, rows_v, src_v, dst_v, acc, sem):
    cid = lax.axis_index("c")
    sid = lax.axis_index("s")
    wid = sid * NC + cid

    zeros = jnp.zeros((16,), jnp.float32)

    def zbody(i, carry):
        rows_v[i // 8, pl.ds((i % 8) * 16, 16)] = zeros
        return carry

    lax.fori_loop(0, CHUNK * 8, zbody, 0)
    base = sid * SLAB
    for k in range(SLAB // CHUNK):
        pltpu.sync_copy(rows_v, acc.at[pl.ds(base + k * CHUNK, CHUNK)])
    plsc.subcore_barrier()

    n_my = jnp.where(wid < N_CHUNKS % NW, N_CHUNKS // NW + 1, N_CHUNKS // NW)

    def body(t, carry):
        c = wid + t * NW
        pltpu.sync_copy(edges_hbm.at[0, pl.ds(c * CHUNK, CHUNK)], src_v)
        pltpu.sync_copy(edges_hbm.at[1, pl.ds(c * CHUNK, CHUNK)], dst_v)
        pltpu.async_copy(g_hbm.at[src_v], rows_v, sem).wait()
        pltpu.sync_copy(rows_v, acc.at[dst_v], add=True)
        return carry

    lax.fori_loop(0, n_my, body, 0)
    plsc.subcore_barrier()

    @pl.when(sid < NS - 1)
    def _():
        pltpu.sync_copy(acc.at[pl.ds(base, SLAB)],
                        out_hbm.at[cid, pl.ds(base, SLAB)])

    @pl.when(sid == NS - 1)
    def _():
        last = (NS - 1) * SLAB
        pltpu.sync_copy(acc.at[pl.ds(last, N - last)],
                        out_hbm.at[cid, pl.ds(last, N - last)])


def _prep_body(x_ref, w_ref, hist_ref, g_ref, dis_ref):
    deg = 1.0 + jnp.sum(hist_ref[...], axis=1, keepdims=True)
    dis = lax.rsqrt(deg)
    dis_ref[...] = dis
    xw = jnp.dot(x_ref[...], w_ref[...], preferred_element_type=jnp.float32)
    g_ref[...] = xw * dis


def _prep(x, w1, hist):
    return pl.pallas_call(
        _prep_body,
        grid=(N_BLOCKS,),
        in_specs=[
            pl.BlockSpec((R, D), lambda i: (i, 0)),
            pl.BlockSpec((D, D), lambda i: (0, 0)),
            pl.BlockSpec((R, NC), lambda i: (i, 0)),
        ],
        out_specs=[
            pl.BlockSpec((R, D), lambda i: (i, 0)),
            pl.BlockSpec((R, 1), lambda i: (i, 0)),
        ],
        out_shape=[
            jax.ShapeDtypeStruct((N, D), jnp.float32),
            jax.ShapeDtypeStruct((N, 1), jnp.float32),
        ],
    )(x, w1, hist)


def _mid_body(a0_ref, a1_ref, g_ref, dis_ref, b_ref, w_ref, gout_ref):
    dis = dis_ref[...]
    h = dis * (a0_ref[...] + a1_ref[...] + g_ref[...]) + b_ref[...]
    h = jnp.maximum(h, 0.0)
    hw = jnp.dot(h, w_ref[...], preferred_element_type=jnp.float32)
    gout_ref[...] = hw * dis


def _mid(a0, a1, g, dis, b, w):
    return pl.pallas_call(
        _mid_body,
        grid=(N_BLOCKS,),
        in_specs=[
            pl.BlockSpec((R, D), lambda i: (i, 0)),
            pl.BlockSpec((R, D), lambda i: (i, 0)),
            pl.BlockSpec((R, D), lambda i: (i, 0)),
            pl.BlockSpec((R, 1), lambda i: (i, 0)),
            pl.BlockSpec((1, D), lambda i: (0, 0)),
            pl.BlockSpec((D, D), lambda i: (0, 0)),
        ],
        out_specs=pl.BlockSpec((R, D), lambda i: (i, 0)),
        out_shape=jax.ShapeDtypeStruct((N, D), jnp.float32),
    )(a0, a1, g, dis, b, w)


def _final_body(a0_ref, a1_ref, g_ref, dis_ref, b_ref, batch_ref, wl_ref,
                bl_ref, out_ref, sums, cnts):
    i = pl.program_id(0)

    @pl.when(i == 0)
    def _():
        sums[...] = jnp.zeros_like(sums)
        cnts[...] = jnp.zeros_like(cnts)

    h = dis_ref[...] * (a0_ref[...] + a1_ref[...] + g_ref[...]) + b_ref[...]
    bt = batch_ref[...][:, 0]
    gid = lax.broadcasted_iota(jnp.int32, (G_OUT, R), 0)
    onehot = (gid == bt[None, :]).astype(jnp.float32)
    sums[...] += jnp.dot(onehot, h, preferred_element_type=jnp.float32)
    cnts[...] += jnp.sum(onehot, axis=1)[:, None]

    @pl.when(i == N_BLOCKS - 1)
    def _():
        pooled = sums[...] / jnp.maximum(cnts[...], 1.0)
        out_ref[...] = (
            jnp.dot(pooled, wl_ref[...], preferred_element_type=jnp.float32)
            + bl_ref[...]
        )


def _final(a0, a1, g, dis, b, batch2d, wl, bl):
    return pl.pallas_call(
        _final_body,
        grid=(N_BLOCKS,),
        in_specs=[
            pl.BlockSpec((R, D), lambda i: (i, 0)),
            pl.BlockSpec((R, D), lambda i: (i, 0)),
            pl.BlockSpec((R, D), lambda i: (i, 0)),
            pl.BlockSpec((R, 1), lambda i: (i, 0)),
            pl.BlockSpec((1, D), lambda i: (0, 0)),
            pl.BlockSpec((R, 1), lambda i: (i, 0)),
            pl.BlockSpec((D, G_OUT), lambda i: (0, 0)),
            pl.BlockSpec((1, G_OUT), lambda i: (0, 0)),
        ],
        out_specs=pl.BlockSpec((G_OUT, G_OUT), lambda i: (0, 0)),
        out_shape=jax.ShapeDtypeStruct((G_OUT, G_OUT), jnp.float32),
        scratch_shapes=[
            pltpu.VMEM((G_OUT, D), jnp.float32),
            pltpu.VMEM((G_OUT, 1), jnp.float32),
        ],
    )(a0, a1, g, dis, b, batch2d, wl, bl)


def kernel(x, edge_index, batch, layers, W1, b1, W2, b2, W3, b3, Wl, bl):
    x = x.reshape(-1, x.shape[-1]).astype(jnp.float32)
    batch2d = batch.reshape(N, 1)
    b1r = b1.reshape(1, D)
    b2r = b2.reshape(1, D)
    b3r = b3.reshape(1, D)
    blr = bl.reshape(1, G_OUT)

    hist = _deg(edge_index).T
    g1, dis = _prep(x, W1, hist)
    p1 = _prop(g1, edge_index)
    g2 = _mid(p1[0], p1[1], g1, dis, b1r, W2)
    p2 = _prop(g2, edge_index)
    g3 = _mid(p2[0], p2[1], g2, dis, b2r, W3)
    p3 = _prop(g3, edge_index)
    return _final(p3[0], p3[1], g3, dis, b3r, batch2d, Wl, blr)


# speedup vs baseline: 33.4209x; 1.7212x over previous
import functools

import jax
import jax.numpy as jnp
from jax import lax
from jax.experimental import pallas as pl
from jax.experimental.pallas import tpu as pltpu
from jax.experimental.pallas import tpu_sc as plsc

N = 10000
D = 128
E = 320000
G_OUT = 64
CHUNK = 128
N_CHUNKS = E // CHUNK
NC, NS = 2, 16
NW = NC * NS
ROWS_PER_TILE = N // NS
R = 2000
N_BLOCKS = N // R

_mesh = plsc.VectorSubcoreMesh(core_axis_name="c", subcore_axis_name="s")


N_PAD = 10240
SLAB = N_PAD // NS


@functools.partial(
    pl.kernel,
    out_type=jax.ShapeDtypeStruct((NC, N_PAD), jnp.float32),
    mesh=_mesh,
    scratch_types=[
        pltpu.VMEM((CHUNK,), jnp.int32),
        pltpu.VMEM((CHUNK,), jnp.float32),
        pltpu.VMEM_SHARED((N_PAD,), jnp.float32),
    ],
)
def _deg(edges_hbm, out_hbm, idx_v, val_v, hacc):
    cid = lax.axis_index("c")
    sid = lax.axis_index("s")
    wid = sid * NC + cid

    zeros = jnp.zeros((16,), jnp.float32)

    def zbody(i, carry):
        val_v[pl.ds(i * 16, 16)] = zeros
        return carry

    lax.fori_loop(0, CHUNK // 16, zbody, 0)
    base = sid * SLAB
    for k in range(SLAB // CHUNK):
        pltpu.sync_copy(val_v, hacc.at[pl.ds(base + k * CHUNK, CHUNK)])
    plsc.subcore_barrier()

    ones = jnp.ones((16,), jnp.float32)

    def obody(i, carry):
        val_v[pl.ds(i * 16, 16)] = ones
        return carry

    lax.fori_loop(0, CHUNK // 16, obody, 0)

    n_my = jnp.where(wid < N_CHUNKS % NW, N_CHUNKS // NW + 1, N_CHUNKS // NW)

    def body(t, carry):
        c = wid + t * NW
        pltpu.sync_copy(edges_hbm.at[1, pl.ds(c * CHUNK, CHUNK)], idx_v)
        pltpu.sync_copy(val_v, hacc.at[idx_v], add=True)
        return carry

    lax.fori_loop(0, n_my, body, 0)
    plsc.subcore_barrier()
    pltpu.sync_copy(hacc.at[pl.ds(base, SLAB)],
                    out_hbm.at[cid, pl.ds(base, SLAB)])


@functools.partial(
    pl.kernel,
    out_type=jax.ShapeDtypeStruct((NC, N, D), jnp.float32),
    mesh=_mesh,
    scratch_types=[
        pltpu.VMEM((CHUNK, D), jnp.float32),
        pltpu.VMEM((CHUNK, D), jnp.float32),
        pltpu.VMEM((2, CHUNK), jnp.int32),
        pltpu.VMEM((2, CHUNK), jnp.int32),
        pltpu.VMEM_SHARED((N_PAD, D), jnp.float32),
        pltpu.SemaphoreType.DMA,
        pltpu.SemaphoreType.DMA,
        pltpu.SemaphoreType.DMA,
        pltpu.SemaphoreType.DMA,
    ],
)
def _prop(g_hbm, edges_hbm, out_hbm, rows_a, rows_b, eidx_a, eidx_b, acc,
          semi_a, semi_b, semg_a, semg_b):
    cid = lax.axis_index("c")
    sid = lax.axis_index("s")
    wid = sid * NC + cid

    zeros = jnp.zeros((16,), jnp.float32)

    def zbody(i, carry):
        rows_a[i // 8, pl.ds((i % 8) * 16, 16)] = zeros
        return carry

    lax.fori_loop(0, CHUNK * 8, zbody, 0)
    base = sid * SLAB
    for k in range(SLAB // CHUNK):
        pltpu.sync_copy(rows_a, acc.at[pl.ds(base + k * CHUNK, CHUNK)])
    plsc.subcore_barrier()

    n_my = jnp.where(wid < N_CHUNKS % NW, N_CHUNKS // NW + 1, N_CHUNKS // NW)
    n_lo = N_CHUNKS // NW

    def chunk_of(t):
        return wid + t * NW

    def load_idx(t, eidx, semi):
        pltpu.async_copy(edges_hbm.at[:, pl.ds(chunk_of(t) * CHUNK, CHUNK)],
                         eidx, semi)

    def gather(t, eidx, rows, semi, semg):
        pltpu.make_async_copy(edges_hbm.at[:, pl.ds(0, CHUNK)], eidx,
                              semi).wait()
        pltpu.async_copy(g_hbm.at[eidx.at[0]], rows, semg)

    def scatter(rows, eidx):
        pltpu.sync_copy(rows, acc.at[eidx.at[1]], add=True)

    load_idx(0, eidx_a, semi_a)
    gather(0, eidx_a, rows_a, semi_a, semg_a)
    load_idx(1, eidx_b, semi_b)

    def pair(u, carry):
        ta = 2 * u
        pltpu.make_async_copy(g_hbm.at[eidx_a.at[0]], rows_a, semg_a).wait()
        gather(ta + 1, eidx_b, rows_b, semi_b, semg_b)
        scatter(rows_a, eidx_a)

        @pl.when(ta + 2 < n_my)
        def _():
            load_idx(ta + 2, eidx_a, semi_a)
            gather(ta + 2, eidx_a, rows_a, semi_a, semg_a)

        pltpu.make_async_copy(g_hbm.at[eidx_b.at[0]], rows_b, semg_b).wait()
        scatter(rows_b, eidx_b)

        @pl.when(ta + 3 < n_my)
        def _():
            load_idx(ta + 3, eidx_b, semi_b)

        return carry

    lax.fori_loop(0, n_lo // 2, pair, 0)

    @pl.when(n_my > n_lo)
    def _():
        pltpu.make_async_copy(g_hbm.at[eidx_a.at[0]], rows_a, semg_a).wait()
        scatter(rows_a, eidx_a)

    plsc.subcore_barrier()

    @pl.when(sid < NS - 1)
    def _():
        pltpu.sync_copy(acc.at[pl.ds(base, SLAB)],
                        out_hbm.at[cid, pl.ds(base, SLAB)])

    @pl.when(sid == NS - 1)
    def _():
        last = (NS - 1) * SLAB
        pltpu.sync_copy(acc.at[pl.ds(last, N - last)],
                        out_hbm.at[cid, pl.ds(last, N - last)])


def _prep_body(x_ref, w_ref, hist_ref, g_ref, dis_ref):
    deg = 1.0 + jnp.sum(hist_ref[...], axis=1, keepdims=True)
    dis = lax.rsqrt(deg)
    dis_ref[...] = dis
    xw = jnp.dot(x_ref[...], w_ref[...], preferred_element_type=jnp.float32)
    g_ref[...] = xw * dis


def _prep(x, w1, hist):
    return pl.pallas_call(
        _prep_body,
        grid=(N_BLOCKS,),
        in_specs=[
            pl.BlockSpec((R, D), lambda i: (i, 0)),
            pl.BlockSpec((D, D), lambda i: (0, 0)),
            pl.BlockSpec((R, NC), lambda i: (i, 0)),
        ],
        out_specs=[
            pl.BlockSpec((R, D), lambda i: (i, 0)),
            pl.BlockSpec((R, 1), lambda i: (i, 0)),
        ],
        out_shape=[
            jax.ShapeDtypeStruct((N, D), jnp.float32),
            jax.ShapeDtypeStruct((N, 1), jnp.float32),
        ],
    )(x, w1, hist)


def _mid_body(a0_ref, a1_ref, g_ref, dis_ref, b_ref, w_ref, gout_ref):
    dis = dis_ref[...]
    h = dis * (a0_ref[...] + a1_ref[...] + g_ref[...]) + b_ref[...]
    h = jnp.maximum(h, 0.0)
    hw = jnp.dot(h, w_ref[...], preferred_element_type=jnp.float32)
    gout_ref[...] = hw * dis


def _mid(a0, a1, g, dis, b, w):
    return pl.pallas_call(
        _mid_body,
        grid=(N_BLOCKS,),
        in_specs=[
            pl.BlockSpec((R, D), lambda i: (i, 0)),
            pl.BlockSpec((R, D), lambda i: (i, 0)),
            pl.BlockSpec((R, D), lambda i: (i, 0)),
            pl.BlockSpec((R, 1), lambda i: (i, 0)),
            pl.BlockSpec((1, D), lambda i: (0, 0)),
            pl.BlockSpec((D, D), lambda i: (0, 0)),
        ],
        out_specs=pl.BlockSpec((R, D), lambda i: (i, 0)),
        out_shape=jax.ShapeDtypeStruct((N, D), jnp.float32),
    )(a0, a1, g, dis, b, w)


def _final_body(a0_ref, a1_ref, g_ref, dis_ref, b_ref, batch_ref, wl_ref,
                bl_ref, out_ref, sums, cnts):
    i = pl.program_id(0)

    @pl.when(i == 0)
    def _():
        sums[...] = jnp.zeros_like(sums)
        cnts[...] = jnp.zeros_like(cnts)

    h = dis_ref[...] * (a0_ref[...] + a1_ref[...] + g_ref[...]) + b_ref[...]
    bt = batch_ref[...][:, 0]
    gid = lax.broadcasted_iota(jnp.int32, (G_OUT, R), 0)
    onehot = (gid == bt[None, :]).astype(jnp.float32)
    sums[...] += jnp.dot(onehot, h, preferred_element_type=jnp.float32)
    cnts[...] += jnp.sum(onehot, axis=1)[:, None]

    @pl.when(i == N_BLOCKS - 1)
    def _():
        pooled = sums[...] / jnp.maximum(cnts[...], 1.0)
        out_ref[...] = (
            jnp.dot(pooled, wl_ref[...], preferred_element_type=jnp.float32)
            + bl_ref[...]
        )


def _final(a0, a1, g, dis, b, batch2d, wl, bl):
    return pl.pallas_call(
        _final_body,
        grid=(N_BLOCKS,),
        in_specs=[
            pl.BlockSpec((R, D), lambda i: (i, 0)),
            pl.BlockSpec((R, D), lambda i: (i, 0)),
            pl.BlockSpec((R, D), lambda i: (i, 0)),
            pl.BlockSpec((R, 1), lambda i: (i, 0)),
            pl.BlockSpec((1, D), lambda i: (0, 0)),
            pl.BlockSpec((R, 1), lambda i: (i, 0)),
            pl.BlockSpec((D, G_OUT), lambda i: (0, 0)),
            pl.BlockSpec((1, G_OUT), lambda i: (0, 0)),
        ],
        out_specs=pl.BlockSpec((G_OUT, G_OUT), lambda i: (0, 0)),
        out_shape=jax.ShapeDtypeStruct((G_OUT, G_OUT), jnp.float32),
        scratch_shapes=[
            pltpu.VMEM((G_OUT, D), jnp.float32),
            pltpu.VMEM((G_OUT, 1), jnp.float32),
        ],
    )(a0, a1, g, dis, b, batch2d, wl, bl)


def kernel(x, edge_index, batch, layers, W1, b1, W2, b2, W3, b3, Wl, bl):
    x = x.reshape(-1, x.shape[-1]).astype(jnp.float32)
    batch2d = batch.reshape(N, 1)
    b1r = b1.reshape(1, D)
    b2r = b2.reshape(1, D)
    b3r = b3.reshape(1, D)
    blr = bl.reshape(1, G_OUT)

    hist = _deg(edge_index).T
    g1, dis = _prep(x, W1, hist)
    p1 = _prop(g1, edge_index)
    g2 = _mid(p1[0], p1[1], g1, dis, b1r, W2)
    p2 = _prop(g2, edge_index)
    g3 = _mid(p2[0], p2[1], g2, dis, b2r, W3)
    p3 = _prop(g3, edge_index)
    return _final(p3[0], p3[1], g3, dis, b3r, batch2d, Wl, blr)


# speedup vs baseline: 38.8084x; 1.1612x over previous
import functools

import jax
import jax.numpy as jnp
from jax import lax
from jax.experimental import pallas as pl
from jax.experimental.pallas import tpu as pltpu
from jax.experimental.pallas import tpu_sc as plsc

N = 10000
D = 128
E = 320000
G_OUT = 64
CHUNK = 128
N_CHUNKS = E // CHUNK
NC, NS = 2, 16
NW = NC * NS
CPW = 80
R = 2000
N_BLOCKS = N // R
N_PAD = 10240
SLAB = N_PAD // NS

_mesh = plsc.VectorSubcoreMesh(core_axis_name="c", subcore_axis_name="s")


def _span(wid):
    start = 80 * wid - 8 * jnp.maximum(0, wid - 25)
    n = jnp.where(wid < 25, 80, jnp.where(wid < 31, 72, 68))
    return start, n


@functools.partial(
    pl.kernel,
    out_type=jax.ShapeDtypeStruct((NC, N_PAD), jnp.float32),
    mesh=_mesh,
    scratch_types=[
        pltpu.VMEM((CPW, CHUNK), jnp.int32),
        pltpu.VMEM((CHUNK,), jnp.float32),
        pltpu.VMEM_SHARED((N_PAD,), jnp.float32),
        pltpu.SemaphoreType.DMA,
    ],
)
def _deg(e3_hbm, out_hbm, idx_v, val_v, hacc, sem):
    cid = lax.axis_index("c")
    sid = lax.axis_index("s")
    wid = sid * NC + cid
    start, n_my = _span(wid)

    zeros = jnp.zeros((16,), jnp.float32)

    def zbody(i, carry):
        val_v[pl.ds(i * 16, 16)] = zeros
        return carry

    lax.fori_loop(0, CHUNK // 16, zbody, 0)
    base = sid * SLAB
    for k in range(SLAB // CHUNK):
        pltpu.sync_copy(val_v, hacc.at[pl.ds(base + k * CHUNK, CHUNK)])
    plsc.subcore_barrier()

    ones = jnp.ones((16,), jnp.float32)

    def obody(i, carry):
        val_v[pl.ds(i * 16, 16)] = ones
        return carry

    lax.fori_loop(0, CHUNK // 16, obody, 0)

    @pl.when(n_my == 80)
    def _():
        pltpu.sync_copy(e3_hbm.at[1, pl.ds(start, 80)], idx_v)

    @pl.when(n_my < 80)
    def _():
        pltpu.sync_copy(e3_hbm.at[1, pl.ds(start, 72)],
                        idx_v.at[pl.ds(0, 72)])

    def fire(i, carry):
        pltpu.async_copy(val_v, hacc.at[idx_v.at[i]], sem, add=True)
        return carry

    lax.fori_loop(0, n_my, fire, 0)

    def drain(i, carry):
        pltpu.make_async_copy(val_v, hacc.at[idx_v.at[0]], sem).wait()
        return carry

    lax.fori_loop(0, n_my, drain, 0)
    plsc.subcore_barrier()
    pltpu.sync_copy(hacc.at[pl.ds(base, SLAB)],
                    out_hbm.at[cid, pl.ds(base, SLAB)])


@functools.partial(
    pl.kernel,
    out_type=jax.ShapeDtypeStruct((NC, N, D), jnp.float32),
    mesh=_mesh,
    scratch_types=[
        pltpu.VMEM((2, CPW // 2, CHUNK), jnp.int32),
        pltpu.VMEM((CHUNK, D), jnp.float32),
        pltpu.VMEM((CHUNK, D), jnp.float32),
        pltpu.VMEM_SHARED((N_PAD, D), jnp.float32),
        pltpu.SemaphoreType.DMA,
        pltpu.SemaphoreType.DMA,
    ],
)
def _prop(g_hbm, e3_hbm, out_hbm, eidx, rows0, rows1, acc, sem0, sem1):
    cid = lax.axis_index("c")
    sid = lax.axis_index("s")
    wid = sid * NC + cid
    start, n_my = _span(wid)
    rows = (rows0, rows1)
    sems = (sem0, sem1)

    zeros = jnp.zeros((16,), jnp.float32)

    def zbody(i, carry):
        rows0[i // 8, pl.ds((i % 8) * 16, 16)] = zeros
        return carry

    lax.fori_loop(0, CHUNK * 8, zbody, 0)
    base = sid * SLAB
    for k in range(SLAB // CHUNK):
        pltpu.sync_copy(rows0, acc.at[pl.ds(base + k * CHUNK, CHUNK)])
    plsc.subcore_barrier()

    def gather(t, slot):
        pltpu.async_copy(g_hbm.at[eidx.at[0, t]], rows[slot], sems[slot])

    def gwait(slot):
        pltpu.make_async_copy(g_hbm.at[eidx.at[0, 0]], rows[slot],
                              sems[slot]).wait()

    def scatter(t, slot):
        pltpu.sync_copy(rows[slot], acc.at[eidx.at[1, t]], add=True)

    def run_phase(p_start, nt):
        @pl.when(nt == 40)
        def _():
            pltpu.sync_copy(e3_hbm.at[:, pl.ds(p_start, 40)], eidx)

        @pl.when(nt < 40)
        def _():
            pltpu.sync_copy(e3_hbm.at[:, pl.ds(p_start, 32)],
                            eidx.at[:, pl.ds(0, 32)])

        gather(0, 0)

        def pair(u, carry):
            tl = 2 * u
            gwait(0)
            gather(tl + 1, 1)
            scatter(tl, 0)

            @pl.when(tl + 2 < nt)
            def _():
                gather(tl + 2, 0)

            gwait(1)
            scatter(tl + 1, 1)
            return carry

        lax.fori_loop(0, nt // 2, pair, 0)

    run_phase(start, jnp.int32(40))
    run_phase(start + 40, n_my - 40)
    plsc.subcore_barrier()

    @pl.when(sid < NS - 1)
    def _():
        pltpu.sync_copy(acc.at[pl.ds(base, SLAB)],
                        out_hbm.at[cid, pl.ds(base, SLAB)])

    @pl.when(sid == NS - 1)
    def _():
        last = (NS - 1) * SLAB
        pltpu.sync_copy(acc.at[pl.ds(last, N - last)],
                        out_hbm.at[cid, pl.ds(last, N - last)])


def _prep_body(x_ref, w_ref, hist_ref, g_ref, dis_ref):
    deg = 1.0 + jnp.sum(hist_ref[...], axis=1, keepdims=True)
    dis = lax.rsqrt(deg)
    dis_ref[...] = dis
    xw = jnp.dot(x_ref[...], w_ref[...], preferred_element_type=jnp.float32)
    g_ref[...] = xw * dis


def _prep(x, w1, hist):
    return pl.pallas_call(
        _prep_body,
        grid=(N_BLOCKS,),
        in_specs=[
            pl.BlockSpec((R, D), lambda i: (i, 0)),
            pl.BlockSpec((D, D), lambda i: (0, 0)),
            pl.BlockSpec((R, NC), lambda i: (i, 0)),
        ],
        out_specs=[
            pl.BlockSpec((R, D), lambda i: (i, 0)),
            pl.BlockSpec((R, 1), lambda i: (i, 0)),
        ],
        out_shape=[
            jax.ShapeDtypeStruct((N, D), jnp.float32),
            jax.ShapeDtypeStruct((N, 1), jnp.float32),
        ],
    )(x, w1, hist)


def _mid_body(a0_ref, a1_ref, g_ref, dis_ref, b_ref, w_ref, gout_ref):
    dis = dis_ref[...]
    h = dis * (a0_ref[...] + a1_ref[...] + g_ref[...]) + b_ref[...]
    h = jnp.maximum(h, 0.0)
    hw = jnp.dot(h, w_ref[...], preferred_element_type=jnp.float32)
    gout_ref[...] = hw * dis


def _mid(a0, a1, g, dis, b, w):
    return pl.pallas_call(
        _mid_body,
        grid=(N_BLOCKS,),
        in_specs=[
            pl.BlockSpec((R, D), lambda i: (i, 0)),
            pl.BlockSpec((R, D), lambda i: (i, 0)),
            pl.BlockSpec((R, D), lambda i: (i, 0)),
            pl.BlockSpec((R, 1), lambda i: (i, 0)),
            pl.BlockSpec((1, D), lambda i: (0, 0)),
            pl.BlockSpec((D, D), lambda i: (0, 0)),
        ],
        out_specs=pl.BlockSpec((R, D), lambda i: (i, 0)),
        out_shape=jax.ShapeDtypeStruct((N, D), jnp.float32),
    )(a0, a1, g, dis, b, w)


def _final_body(a0_ref, a1_ref, g_ref, dis_ref, b_ref, batch_ref, wl_ref,
                bl_ref, out_ref, sums, cnts):
    i = pl.program_id(0)

    @pl.when(i == 0)
    def _():
        sums[...] = jnp.zeros_like(sums)
        cnts[...] = jnp.zeros_like(cnts)

    h = dis_ref[...] * (a0_ref[...] + a1_ref[...] + g_ref[...]) + b_ref[...]
    bt = batch_ref[...][:, 0]
    gid = lax.broadcasted_iota(jnp.int32, (G_OUT, R), 0)
    onehot = (gid == bt[None, :]).astype(jnp.float32)
    sums[...] += jnp.dot(onehot, h, preferred_element_type=jnp.float32)
    cnts[...] += jnp.sum(onehot, axis=1)[:, None]

    @pl.when(i == N_BLOCKS - 1)
    def _():
        pooled = sums[...] / jnp.maximum(cnts[...], 1.0)
        out_ref[...] = (
            jnp.dot(pooled, wl_ref[...], preferred_element_type=jnp.float32)
            + bl_ref[...]
        )


def _final(a0, a1, g, dis, b, batch2d, wl, bl):
    return pl.pallas_call(
        _final_body,
        grid=(N_BLOCKS,),
        in_specs=[
            pl.BlockSpec((R, D), lambda i: (i, 0)),
            pl.BlockSpec((R, D), lambda i: (i, 0)),
            pl.BlockSpec((R, D), lambda i: (i, 0)),
            pl.BlockSpec((R, 1), lambda i: (i, 0)),
            pl.BlockSpec((1, D), lambda i: (0, 0)),
            pl.BlockSpec((R, 1), lambda i: (i, 0)),
            pl.BlockSpec((D, G_OUT), lambda i: (0, 0)),
            pl.BlockSpec((1, G_OUT), lambda i: (0, 0)),
        ],
        out_specs=pl.BlockSpec((G_OUT, G_OUT), lambda i: (0, 0)),
        out_shape=jax.ShapeDtypeStruct((G_OUT, G_OUT), jnp.float32),
        scratch_shapes=[
            pltpu.VMEM((G_OUT, D), jnp.float32),
            pltpu.VMEM((G_OUT, 1), jnp.float32),
        ],
    )(a0, a1, g, dis, b, batch2d, wl, bl)


def kernel(x, edge_index, batch, layers, W1, b1, W2, b2, W3, b3, Wl, bl):
    x = x.reshape(-1, x.shape[-1]).astype(jnp.float32)
    e3 = edge_index.reshape(2, N_CHUNKS, CHUNK)
    batch2d = batch.reshape(N, 1)
    b1r = b1.reshape(1, D)
    b2r = b2.reshape(1, D)
    b3r = b3.reshape(1, D)
    blr = bl.reshape(1, G_OUT)

    hist = _deg(e3).T
    g1, dis = _prep(x, W1, hist)
    p1 = _prop(g1, e3)
    g2 = _mid(p1[0], p1[1], g1, dis, b1r, W2)
    p2 = _prop(g2, e3)
    g3 = _mid(p2[0], p2[1], g2, dis, b2r, W3)
    p3 = _prop(g3, e3)
    return _final(p3[0], p3[1], g3, dis, b3r, batch2d, Wl, blr)


# speedup vs baseline: 39.5224x; 1.0184x over previous
import functools

import jax
import jax.numpy as jnp
from jax import lax
from jax.experimental import pallas as pl
from jax.experimental.pallas import tpu as pltpu
from jax.experimental.pallas import tpu_sc as plsc

N = 10000
D = 128
E = 320000
G_OUT = 64
CHUNK = 128
N_CHUNKS = E // CHUNK
NC, NS = 2, 16
NW = NC * NS
CPW = 80
R = 2000
N_BLOCKS = N // R
N_PAD = 10240
SLAB = N_PAD // NS

_mesh = plsc.VectorSubcoreMesh(core_axis_name="c", subcore_axis_name="s")


def _span(wid):
    start = 80 * wid - 8 * jnp.maximum(0, wid - 25)
    n = jnp.where(wid < 25, 80, jnp.where(wid < 31, 72, 68))
    return start, n


@functools.partial(
    pl.kernel,
    out_type=jax.ShapeDtypeStruct((NC, N_PAD), jnp.float32),
    mesh=_mesh,
    scratch_types=[
        pltpu.VMEM((CPW, CHUNK), jnp.int32),
        pltpu.VMEM((CHUNK,), jnp.float32),
        pltpu.VMEM_SHARED((N_PAD,), jnp.float32),
        pltpu.SemaphoreType.DMA,
    ],
)
def _deg(e3_hbm, out_hbm, idx_v, val_v, hacc, sem):
    cid = lax.axis_index("c")
    sid = lax.axis_index("s")
    wid = sid * NC + cid
    start, n_my = _span(wid)

    zeros = jnp.zeros((16,), jnp.float32)

    def zbody(i, carry):
        val_v[pl.ds(i * 16, 16)] = zeros
        return carry

    lax.fori_loop(0, CHUNK // 16, zbody, 0)
    base = sid * SLAB
    for k in range(SLAB // CHUNK):
        pltpu.sync_copy(val_v, hacc.at[pl.ds(base + k * CHUNK, CHUNK)])
    plsc.subcore_barrier()

    ones = jnp.ones((16,), jnp.float32)

    def obody(i, carry):
        val_v[pl.ds(i * 16, 16)] = ones
        return carry

    lax.fori_loop(0, CHUNK // 16, obody, 0)

    @pl.when(n_my == 80)
    def _():
        pltpu.sync_copy(e3_hbm.at[1, pl.ds(start, 80)], idx_v)

    @pl.when(n_my < 80)
    def _():
        pltpu.sync_copy(e3_hbm.at[1, pl.ds(start, 72)],
                        idx_v.at[pl.ds(0, 72)])

    def fire(i, carry):
        pltpu.async_copy(val_v, hacc.at[idx_v.at[i]], sem, add=True)
        return carry

    lax.fori_loop(0, n_my, fire, 0)

    def drain(i, carry):
        pltpu.make_async_copy(val_v, hacc.at[idx_v.at[0]], sem).wait()
        return carry

    lax.fori_loop(0, n_my, drain, 0)
    plsc.subcore_barrier()
    pltpu.sync_copy(hacc.at[pl.ds(base, SLAB)],
                    out_hbm.at[cid, pl.ds(base, SLAB)])


@functools.partial(
    pl.kernel,
    out_type=jax.ShapeDtypeStruct((NC, N, D), jnp.float32),
    mesh=_mesh,
    scratch_types=[
        pltpu.VMEM((2, CPW // 2, CHUNK), jnp.int32),
        pltpu.VMEM((CHUNK, D), jnp.float32),
        pltpu.VMEM((CHUNK, D), jnp.float32),
        pltpu.VMEM_SHARED((N_PAD, D), jnp.float32),
        pltpu.SemaphoreType.DMA,
        pltpu.SemaphoreType.DMA,
        pltpu.SemaphoreType.DMA,
        pltpu.SemaphoreType.DMA,
    ],
)
def _prop(g_hbm, e3_hbm, out_hbm, eidx, rows0, rows1, acc, sem0a, sem0b,
          sem1a, sem1b):
    cid = lax.axis_index("c")
    sid = lax.axis_index("s")
    wid = sid * NC + cid
    start, n_my = _span(wid)
    rows = (rows0, rows1)
    sems = ((sem0a, sem0b), (sem1a, sem1b))
    H = CHUNK // 2

    zeros = jnp.zeros((16,), jnp.float32)

    def zbody(i, carry):
        rows1[i // 8, pl.ds((i % 8) * 16, 16)] = zeros
        return carry

    lax.fori_loop(0, CHUNK * 8, zbody, 0)
    base = sid * SLAB

    def gather(t, slot):
        pltpu.async_copy(g_hbm.at[eidx.at[0, t, pl.ds(0, H)]],
                         rows[slot].at[pl.ds(0, H)], sems[slot][0])
        pltpu.async_copy(g_hbm.at[eidx.at[0, t, pl.ds(H, H)]],
                         rows[slot].at[pl.ds(H, H)], sems[slot][1])

    def gwait(slot):
        pltpu.make_async_copy(g_hbm.at[eidx.at[0, 0, pl.ds(0, H)]],
                              rows[slot].at[pl.ds(0, H)],
                              sems[slot][0]).wait()
        pltpu.make_async_copy(g_hbm.at[eidx.at[0, 0, pl.ds(0, H)]],
                              rows[slot].at[pl.ds(H, H)],
                              sems[slot][1]).wait()

    def scatter(t, slot):
        pltpu.sync_copy(rows[slot], acc.at[eidx.at[1, t]], add=True)

    def run_phase(p_start, nt, first):
        @pl.when(nt == 40)
        def _():
            pltpu.sync_copy(e3_hbm.at[:, pl.ds(p_start, 40)], eidx)

        @pl.when(nt < 40)
        def _():
            pltpu.sync_copy(e3_hbm.at[:, pl.ds(p_start, 32)],
                            eidx.at[:, pl.ds(0, 32)])

        gather(0, 0)
        if first:
            for k in range(SLAB // CHUNK):
                pltpu.sync_copy(rows1,
                                acc.at[pl.ds(base + k * CHUNK, CHUNK)])
            plsc.subcore_barrier()
        gather(1, 1)

        def pair(u, carry):
            tl = 2 * u
            gwait(0)
            scatter(tl, 0)

            @pl.when(tl + 2 < nt)
            def _():
                gather(tl + 2, 0)

            gwait(1)
            scatter(tl + 1, 1)

            @pl.when(tl + 3 < nt)
            def _():
                gather(tl + 3, 1)

            return carry

        lax.fori_loop(0, nt // 2, pair, 0)

    run_phase(start, jnp.int32(40), True)
    run_phase(start + 40, n_my - 40, False)
    plsc.subcore_barrier()

    @pl.when(sid < NS - 1)
    def _():
        pltpu.sync_copy(acc.at[pl.ds(base, SLAB)],
                        out_hbm.at[cid, pl.ds(base, SLAB)])

    @pl.when(sid == NS - 1)
    def _():
        last = (NS - 1) * SLAB
        pltpu.sync_copy(acc.at[pl.ds(last, N - last)],
                        out_hbm.at[cid, pl.ds(last, N - last)])


def _prep_body(x_ref, w_ref, hist_ref, g_ref, dis_ref):
    deg = 1.0 + jnp.sum(hist_ref[...], axis=1, keepdims=True)
    dis = lax.rsqrt(deg)
    dis_ref[...] = dis
    xw = jnp.dot(x_ref[...], w_ref[...], preferred_element_type=jnp.float32)
    g_ref[...] = xw * dis


def _prep(x, w1, hist):
    return pl.pallas_call(
        _prep_body,
        grid=(N_BLOCKS,),
        in_specs=[
            pl.BlockSpec((R, D), lambda i: (i, 0)),
            pl.BlockSpec((D, D), lambda i: (0, 0)),
            pl.BlockSpec((R, NC), lambda i: (i, 0)),
        ],
        out_specs=[
            pl.BlockSpec((R, D), lambda i: (i, 0)),
            pl.BlockSpec((R, 1), lambda i: (i, 0)),
        ],
        out_shape=[
            jax.ShapeDtypeStruct((N, D), jnp.float32),
            jax.ShapeDtypeStruct((N, 1), jnp.float32),
        ],
    )(x, w1, hist)


def _mid_body(a0_ref, a1_ref, g_ref, dis_ref, b_ref, w_ref, gout_ref):
    dis = dis_ref[...]
    h = dis * (a0_ref[...] + a1_ref[...] + g_ref[...]) + b_ref[...]
    h = jnp.maximum(h, 0.0)
    hw = jnp.dot(h, w_ref[...], preferred_element_type=jnp.float32)
    gout_ref[...] = hw * dis


def _mid(a0, a1, g, dis, b, w):
    return pl.pallas_call(
        _mid_body,
        grid=(N_BLOCKS,),
        in_specs=[
            pl.BlockSpec((R, D), lambda i: (i, 0)),
            pl.BlockSpec((R, D), lambda i: (i, 0)),
            pl.BlockSpec((R, D), lambda i: (i, 0)),
            pl.BlockSpec((R, 1), lambda i: (i, 0)),
            pl.BlockSpec((1, D), lambda i: (0, 0)),
            pl.BlockSpec((D, D), lambda i: (0, 0)),
        ],
        out_specs=pl.BlockSpec((R, D), lambda i: (i, 0)),
        out_shape=jax.ShapeDtypeStruct((N, D), jnp.float32),
    )(a0, a1, g, dis, b, w)


def _final_body(a0_ref, a1_ref, g_ref, dis_ref, b_ref, batch_ref, wl_ref,
                bl_ref, out_ref, sums, cnts):
    i = pl.program_id(0)

    @pl.when(i == 0)
    def _():
        sums[...] = jnp.zeros_like(sums)
        cnts[...] = jnp.zeros_like(cnts)

    h = dis_ref[...] * (a0_ref[...] + a1_ref[...] + g_ref[...]) + b_ref[...]
    bt = batch_ref[...][:, 0]
    gid = lax.broadcasted_iota(jnp.int32, (G_OUT, R), 0)
    onehot = (gid == bt[None, :]).astype(jnp.float32)
    sums[...] += jnp.dot(onehot, h, preferred_element_type=jnp.float32)
    cnts[...] += jnp.sum(onehot, axis=1)[:, None]

    @pl.when(i == N_BLOCKS - 1)
    def _():
        pooled = sums[...] / jnp.maximum(cnts[...], 1.0)
        out_ref[...] = (
            jnp.dot(pooled, wl_ref[...], preferred_element_type=jnp.float32)
            + bl_ref[...]
        )


def _final(a0, a1, g, dis, b, batch2d, wl, bl):
    return pl.pallas_call(
        _final_body,
        grid=(N_BLOCKS,),
        in_specs=[
            pl.BlockSpec((R, D), lambda i: (i, 0)),
            pl.BlockSpec((R, D), lambda i: (i, 0)),
            pl.BlockSpec((R, D), lambda i: (i, 0)),
            pl.BlockSpec((R, 1), lambda i: (i, 0)),
            pl.BlockSpec((1, D), lambda i: (0, 0)),
            pl.BlockSpec((R, 1), lambda i: (i, 0)),
            pl.BlockSpec((D, G_OUT), lambda i: (0, 0)),
            pl.BlockSpec((1, G_OUT), lambda i: (0, 0)),
        ],
        out_specs=pl.BlockSpec((G_OUT, G_OUT), lambda i: (0, 0)),
        out_shape=jax.ShapeDtypeStruct((G_OUT, G_OUT), jnp.float32),
        scratch_shapes=[
            pltpu.VMEM((G_OUT, D), jnp.float32),
            pltpu.VMEM((G_OUT, 1), jnp.float32),
        ],
    )(a0, a1, g, dis, b, batch2d, wl, bl)


def kernel(x, edge_index, batch, layers, W1, b1, W2, b2, W3, b3, Wl, bl):
    x = x.reshape(-1, x.shape[-1]).astype(jnp.float32)
    e3 = edge_index.reshape(2, N_CHUNKS, CHUNK)
    batch2d = batch.reshape(N, 1)
    b1r = b1.reshape(1, D)
    b2r = b2.reshape(1, D)
    b3r = b3.reshape(1, D)
    blr = bl.reshape(1, G_OUT)

    hist = _deg(e3).T
    g1, dis = _prep(x, W1, hist)
    p1 = _prop(g1, e3)
    g2 = _mid(p1[0], p1[1], g1, dis, b1r, W2)
    p2 = _prop(g2, e3)
    g3 = _mid(p2[0], p2[1], g2, dis, b2r, W3)
    p3 = _prop(g3, e3)
    return _final(p3[0], p3[1], g3, dis, b3r, batch2d, Wl, blr)
